# baseline (speedup 1.0000x reference)
import numpy as np
import jax
import jax.numpy as jnp
from jax.experimental import pallas as pl
from jax.experimental.pallas import tpu as pltpu

N_AGENTS = 32
OBS_SIZE = 400
AL_OFFSET = 194
NF_AL = 6
BB = 64
ROWS = BB * N_AGENTS
SUB = 128
NSUB = ROWS // SUB


def _build_consts():
    selr = np.zeros((OBS_SIZE, SUB), np.float32)
    sell = np.zeros((OBS_SIZE, SUB), np.float32)
    for bb in range(SUB // N_AGENTS):
        for j in range(N_AGENTS):
            if j <= N_AGENTS - 2:
                selr[AL_OFFSET + NF_AL * j, bb * N_AGENTS + j] = 1.0
            if j >= 1:
                sell[AL_OFFSET + NF_AL * (j - 1), bb * N_AGENTS + j] = 1.0
    r = np.arange(SUB)[:, None]
    c = np.arange(SUB)[None, :]
    tri = ((r % N_AGENTS) > (c % N_AGENTS)).astype(np.float32)
    valid = ((r // N_AGENTS) == (c // N_AGENTS)) & (r != c)
    validf = valid.astype(np.float32)
    base = np.where(valid, np.float32(-9999.0), np.float32(-np.inf)).astype(np.float32)
    return selr, sell, tri, validf, base


def _attn_kernel(obs_ref, wq_ref, wk_ref, wv_ref, bq_ref, bk_ref, bv_ref,
                 selr_ref, sell_ref, tri_ref, validf_ref, base_ref, out_ref):
    x3 = obs_ref[...]
    x = x3.reshape(ROWS, OBS_SIZE)

    q = jnp.dot(x, wq_ref[...], preferred_element_type=jnp.float32) + bq_ref[...]
    k = jnp.dot(x, wk_ref[...], preferred_element_type=jnp.float32) + bk_ref[...]
    v = jnp.dot(x, wv_ref[...], preferred_element_type=jnp.float32) + bv_ref[...]

    ones = (x == 1.0).astype(jnp.float32)
    tri = tri_ref[...] > 0.5
    validf = validf_ref[...]
    base = base_ref[...]

    envs = []
    for sb in range(NSUB):
        sl = slice(sb * SUB, (sb + 1) * SUB)
        qs, ks, vs, os_ = q[sl], k[sl], v[sl], ones[sl]
        padr = jnp.dot(os_, selr_ref[...], preferred_element_type=jnp.float32)
        padl = jnp.dot(os_, sell_ref[...], preferred_element_type=jnp.float32)
        visf = jnp.where(tri, padr, padl) * validf

        s = jax.lax.dot_general(qs, ks, (((1,), (1,)), ((), ())),
                                preferred_element_type=jnp.float32)
        s = jnp.where(visf > 0.5, s, base)

        m = jnp.max(s, axis=-1, keepdims=True)
        e = jnp.exp(s - m)
        p = e / jnp.sum(e, axis=-1, keepdims=True)
        aw = p * visf

        envs.append(jax.lax.dot_general(aw, vs, (((1,), (0,)), ((), ())),
                                        preferred_element_type=jnp.float32))

    env = jnp.concatenate(envs, axis=0)
    out_ref[...] = jnp.concatenate([x, env], axis=-1).reshape(BB, N_AGENTS, 2 * OBS_SIZE)


def kernel(obs, W, b):
    batch = obs.shape[0]
    wq, wk, wv = W[:, :OBS_SIZE], W[:, OBS_SIZE:2 * OBS_SIZE], W[:, 2 * OBS_SIZE:]
    bq = b[:OBS_SIZE].reshape(1, OBS_SIZE)
    bk = b[OBS_SIZE:2 * OBS_SIZE].reshape(1, OBS_SIZE)
    bv = b[2 * OBS_SIZE:].reshape(1, OBS_SIZE)
    selr, sell, tri, validf, base = (jnp.asarray(a) for a in _build_consts())

    grid = (batch // BB,)
    full2 = lambda i: (0, 0)
    return pl.pallas_call(
        _attn_kernel,
        grid=grid,
        in_specs=[
            pl.BlockSpec((BB, N_AGENTS, OBS_SIZE), lambda i: (i, 0, 0)),
            pl.BlockSpec((OBS_SIZE, OBS_SIZE), full2),
            pl.BlockSpec((OBS_SIZE, OBS_SIZE), full2),
            pl.BlockSpec((OBS_SIZE, OBS_SIZE), full2),
            pl.BlockSpec((1, OBS_SIZE), full2),
            pl.BlockSpec((1, OBS_SIZE), full2),
            pl.BlockSpec((1, OBS_SIZE), full2),
            pl.BlockSpec((OBS_SIZE, SUB), full2),
            pl.BlockSpec((OBS_SIZE, SUB), full2),
            pl.BlockSpec((SUB, SUB), full2),
            pl.BlockSpec((SUB, SUB), full2),
            pl.BlockSpec((SUB, SUB), full2),
        ],
        out_specs=pl.BlockSpec((BB, N_AGENTS, 2 * OBS_SIZE), lambda i: (i, 0, 0)),
        compiler_params=pltpu.CompilerParams(dimension_semantics=("parallel",)),
        out_shape=jax.ShapeDtypeStruct((batch, N_AGENTS, 2 * OBS_SIZE), jnp.float32),
    )(obs, wq, wk, wv, bq, bk, bv, selr, sell, tri, validf, base)


# speedup vs baseline: 1.0271x; 1.0271x over previous
import numpy as np
import jax
import jax.numpy as jnp
from jax.experimental import pallas as pl
from jax.experimental.pallas import tpu as pltpu

N_AGENTS = 32
OBS_SIZE = 400
AL_OFFSET = 194
NF_AL = 6
BB = 32
ROWS = BB * N_AGENTS
SUB = 128
NSUB = ROWS // SUB
VIS_LO = 128
VIS_W = 256


def _build_consts():
    selr = np.zeros((VIS_W, SUB), np.float32)
    sell = np.zeros((VIS_W, SUB), np.float32)
    for bb in range(SUB // N_AGENTS):
        for j in range(N_AGENTS):
            if j <= N_AGENTS - 2:
                selr[AL_OFFSET - VIS_LO + NF_AL * j, bb * N_AGENTS + j] = 1.0
            if j >= 1:
                sell[AL_OFFSET - VIS_LO + NF_AL * (j - 1), bb * N_AGENTS + j] = 1.0
    r = np.arange(SUB)[:, None]
    c = np.arange(SUB)[None, :]
    tri = ((r % N_AGENTS) > (c % N_AGENTS)).astype(np.float32)
    valid = ((r // N_AGENTS) == (c // N_AGENTS)) & (r != c)
    validf = valid.astype(np.float32)
    base = np.where(valid, np.float32(-9999.0), np.float32(-np.inf)).astype(np.float32)
    return selr, sell, tri, validf, base


def _attn_kernel(obs_ref, wq_ref, wk_ref, wv_ref, bq_ref, bk_ref, bv_ref,
                 selr_ref, sell_ref, tri_ref, validf_ref, base_ref, out_ref):
    x3 = obs_ref[...]
    x = x3.reshape(ROWS, OBS_SIZE)

    q = jnp.dot(x, wq_ref[...], preferred_element_type=jnp.float32) + bq_ref[...]
    k = jnp.dot(x, wk_ref[...], preferred_element_type=jnp.float32) + bk_ref[...]
    v = jnp.dot(x, wv_ref[...], preferred_element_type=jnp.float32) + bv_ref[...]

    ones = (x[:, VIS_LO:VIS_LO + VIS_W] == 1.0).astype(jnp.float32)
    tri = tri_ref[...] > 0.5
    validf = validf_ref[...]
    base = base_ref[...]

    envs = []
    for sb in range(NSUB):
        sl = slice(sb * SUB, (sb + 1) * SUB)
        qs, ks, vs, os_ = q[sl], k[sl], v[sl], ones[sl]
        padr = jnp.dot(os_, selr_ref[...], preferred_element_type=jnp.float32)
        padl = jnp.dot(os_, sell_ref[...], preferred_element_type=jnp.float32)
        visf = jnp.where(tri, padr, padl) * validf

        s = jax.lax.dot_general(qs, ks, (((1,), (1,)), ((), ())),
                                preferred_element_type=jnp.float32)
        s = jnp.where(visf > 0.5, s, base)

        m = jnp.max(s, axis=-1, keepdims=True)
        e = jnp.exp(s - m)
        p = e / jnp.sum(e, axis=-1, keepdims=True)
        aw = p * visf

        envs.append(jax.lax.dot_general(aw, vs, (((1,), (0,)), ((), ())),
                                        preferred_element_type=jnp.float32))

    env = jnp.concatenate(envs, axis=0)
    out_ref[...] = jnp.concatenate([x, env], axis=-1).reshape(BB, N_AGENTS, 2 * OBS_SIZE)


def kernel(obs, W, b):
    batch = obs.shape[0]
    wq, wk, wv = W[:, :OBS_SIZE], W[:, OBS_SIZE:2 * OBS_SIZE], W[:, 2 * OBS_SIZE:]
    bq = b[:OBS_SIZE].reshape(1, OBS_SIZE)
    bk = b[OBS_SIZE:2 * OBS_SIZE].reshape(1, OBS_SIZE)
    bv = b[2 * OBS_SIZE:].reshape(1, OBS_SIZE)
    selr, sell, tri, validf, base = (jnp.asarray(a) for a in _build_consts())

    grid = (batch // BB,)
    full2 = lambda i: (0, 0)
    return pl.pallas_call(
        _attn_kernel,
        grid=grid,
        in_specs=[
            pl.BlockSpec((BB, N_AGENTS, OBS_SIZE), lambda i: (i, 0, 0)),
            pl.BlockSpec((OBS_SIZE, OBS_SIZE), full2),
            pl.BlockSpec((OBS_SIZE, OBS_SIZE), full2),
            pl.BlockSpec((OBS_SIZE, OBS_SIZE), full2),
            pl.BlockSpec((1, OBS_SIZE), full2),
            pl.BlockSpec((1, OBS_SIZE), full2),
            pl.BlockSpec((1, OBS_SIZE), full2),
            pl.BlockSpec((VIS_W, SUB), full2),
            pl.BlockSpec((VIS_W, SUB), full2),
            pl.BlockSpec((SUB, SUB), full2),
            pl.BlockSpec((SUB, SUB), full2),
            pl.BlockSpec((SUB, SUB), full2),
        ],
        out_specs=pl.BlockSpec((BB, N_AGENTS, 2 * OBS_SIZE), lambda i: (i, 0, 0)),
        compiler_params=pltpu.CompilerParams(dimension_semantics=("parallel",)),
        out_shape=jax.ShapeDtypeStruct((batch, N_AGENTS, 2 * OBS_SIZE), jnp.float32),
    )(obs, wq, wk, wv, bq, bk, bv, selr, sell, tri, validf, base)


# speedup vs baseline: 1.0647x; 1.0365x over previous
import numpy as np
import jax
import jax.numpy as jnp
from jax.experimental import pallas as pl
from jax.experimental.pallas import tpu as pltpu

N_AGENTS = 32
OBS_SIZE = 400
AL_OFFSET = 194
NF_AL = 6
BB = 32
ROWS = BB * N_AGENTS
SUB = 128
NSUB = ROWS // SUB
VIS_LO = 128
VIS_W = 256


def _build_consts():
    selr = np.zeros((VIS_W, SUB), np.float32)
    sell = np.zeros((VIS_W, SUB), np.float32)
    for bb in range(SUB // N_AGENTS):
        for j in range(N_AGENTS):
            if j <= N_AGENTS - 2:
                selr[AL_OFFSET - VIS_LO + NF_AL * j, bb * N_AGENTS + j] = 1.0
            if j >= 1:
                sell[AL_OFFSET - VIS_LO + NF_AL * (j - 1), bb * N_AGENTS + j] = 1.0
    r = np.arange(SUB)[:, None]
    c = np.arange(SUB)[None, :]
    tri = ((r % N_AGENTS) > (c % N_AGENTS)).astype(np.float32)
    valid = ((r // N_AGENTS) == (c // N_AGENTS)) & (r != c)
    validf = valid.astype(np.float32)
    base = np.where(valid, np.float32(-9999.0), np.float32(-np.inf)).astype(np.float32)
    return selr, sell, tri, validf, base


def _attn_kernel(obs_ref, w_ref, b_ref,
                 selr_ref, sell_ref, tri_ref, validf_ref, base_ref, out_ref):
    x3 = obs_ref[...]
    x = x3.reshape(ROWS, OBS_SIZE)

    y = jnp.dot(x, w_ref[...], preferred_element_type=jnp.float32) + b_ref[...]
    q = y[:, :OBS_SIZE]
    k = y[:, OBS_SIZE:2 * OBS_SIZE]
    v = y[:, 2 * OBS_SIZE:]

    ones = (x[:, VIS_LO:VIS_LO + VIS_W] == 1.0).astype(jnp.float32)
    tri = tri_ref[...] > 0.5
    validf = validf_ref[...]
    base = base_ref[...]

    envs = []
    for sb in range(NSUB):
        sl = slice(sb * SUB, (sb + 1) * SUB)
        qs, ks, vs, os_ = q[sl], k[sl], v[sl], ones[sl]
        padr = jnp.dot(os_, selr_ref[...], preferred_element_type=jnp.float32)
        padl = jnp.dot(os_, sell_ref[...], preferred_element_type=jnp.float32)
        visf = jnp.where(tri, padr, padl) * validf

        s = jax.lax.dot_general(qs, ks, (((1,), (1,)), ((), ())),
                                preferred_element_type=jnp.float32)
        s = jnp.where(visf > 0.5, s, base)

        m = jnp.max(s, axis=-1, keepdims=True)
        e = jnp.exp(s - m)
        p = e / jnp.sum(e, axis=-1, keepdims=True)
        aw = p * visf

        envs.append(jax.lax.dot_general(aw, vs, (((1,), (0,)), ((), ())),
                                        preferred_element_type=jnp.float32))

    env = jnp.concatenate(envs, axis=0)
    out_ref[...] = jnp.concatenate([x, env], axis=-1).reshape(BB, N_AGENTS, 2 * OBS_SIZE)


def kernel(obs, W, b):
    batch = obs.shape[0]
    b2 = b.reshape(1, 3 * OBS_SIZE)
    selr, sell, tri, validf, base = (jnp.asarray(a) for a in _build_consts())

    grid = (batch // BB,)
    full2 = lambda i: (0, 0)
    return pl.pallas_call(
        _attn_kernel,
        grid=grid,
        in_specs=[
            pl.BlockSpec((BB, N_AGENTS, OBS_SIZE), lambda i: (i, 0, 0)),
            pl.BlockSpec((OBS_SIZE, 3 * OBS_SIZE), full2),
            pl.BlockSpec((1, 3 * OBS_SIZE), full2),
            pl.BlockSpec((VIS_W, SUB), full2),
            pl.BlockSpec((VIS_W, SUB), full2),
            pl.BlockSpec((SUB, SUB), full2),
            pl.BlockSpec((SUB, SUB), full2),
            pl.BlockSpec((SUB, SUB), full2),
        ],
        out_specs=pl.BlockSpec((BB, N_AGENTS, 2 * OBS_SIZE), lambda i: (i, 0, 0)),
        compiler_params=pltpu.CompilerParams(dimension_semantics=("parallel",)),
        out_shape=jax.ShapeDtypeStruct((batch, N_AGENTS, 2 * OBS_SIZE), jnp.float32),
    )(obs, W, b2, selr, sell, tri, validf, base)


# speedup vs baseline: 1.1399x; 1.0707x over previous
import numpy as np
import jax
import jax.numpy as jnp
from jax.experimental import pallas as pl
from jax.experimental.pallas import tpu as pltpu

N_AGENTS = 32
OBS_SIZE = 400
AL_OFFSET = 194
NF_AL = 6
BB = 32
ROWS = BB * N_AGENTS
SUB = 128
NSUB = ROWS // SUB
VIS_LO = 128
VIS_W = 256


def _build_consts():
    selr = np.zeros((VIS_W, SUB), np.float32)
    sell = np.zeros((VIS_W, SUB), np.float32)
    for bb in range(SUB // N_AGENTS):
        for j in range(N_AGENTS):
            if j <= N_AGENTS - 2:
                selr[AL_OFFSET - VIS_LO + NF_AL * j, bb * N_AGENTS + j] = 1.0
            if j >= 1:
                sell[AL_OFFSET - VIS_LO + NF_AL * (j - 1), bb * N_AGENTS + j] = 1.0
    r = np.arange(SUB)[:, None]
    c = np.arange(SUB)[None, :]
    tri = ((r % N_AGENTS) > (c % N_AGENTS)).astype(np.float32)
    valid = ((r // N_AGENTS) == (c // N_AGENTS)) & (r != c)
    validf = valid.astype(np.float32)
    base = np.where(valid, np.float32(-9999.0), np.float32(-np.inf)).astype(np.float32)
    return selr, sell, tri, validf, base


def _attn_kernel(obs_ref, w_ref, b_ref,
                 selr_ref, sell_ref, tri_ref, validf_ref, base_ref, out_ref):
    x3 = obs_ref[...]
    x = x3.reshape(ROWS, OBS_SIZE)

    y = jnp.dot(x, w_ref[...], preferred_element_type=jnp.float32) + b_ref[...]
    q = y[:, :OBS_SIZE]
    k = y[:, OBS_SIZE:2 * OBS_SIZE]
    v = y[:, 2 * OBS_SIZE:]

    ones = (x[:, VIS_LO:VIS_LO + VIS_W] == 1.0).astype(jnp.float32)
    tri = tri_ref[...] > 0.5
    validf = validf_ref[...]
    base = base_ref[...]

    svals, visfs = [], []
    for sb in range(NSUB):
        sl = slice(sb * SUB, (sb + 1) * SUB)
        qs, ks, os_ = q[sl], k[sl], ones[sl]
        padr = jnp.dot(os_, selr_ref[...], preferred_element_type=jnp.float32)
        padl = jnp.dot(os_, sell_ref[...], preferred_element_type=jnp.float32)
        visf = jnp.where(tri, padr, padl) * validf

        s = jax.lax.dot_general(qs, ks, (((1,), (1,)), ((), ())),
                                preferred_element_type=jnp.float32)
        svals.append(jnp.where(visf > 0.5, s, base))
        visfs.append(visf)

    s_all = jnp.concatenate(svals, axis=0)
    visf_all = jnp.concatenate(visfs, axis=0)
    m = jnp.max(s_all, axis=-1, keepdims=True)
    e = jnp.exp(s_all - m)
    p = e / jnp.sum(e, axis=-1, keepdims=True)
    aw_all = p * visf_all

    envs = []
    for sb in range(NSUB):
        sl = slice(sb * SUB, (sb + 1) * SUB)
        envs.append(jax.lax.dot_general(aw_all[sl], v[sl], (((1,), (0,)), ((), ())),
                                        preferred_element_type=jnp.float32))

    env = jnp.concatenate(envs, axis=0)
    out_ref[...] = jnp.concatenate([x, env], axis=-1).reshape(BB, N_AGENTS, 2 * OBS_SIZE)


def kernel(obs, W, b):
    batch = obs.shape[0]
    b2 = b.reshape(1, 3 * OBS_SIZE)
    selr, sell, tri, validf, base = (jnp.asarray(a) for a in _build_consts())

    grid = (batch // BB,)
    full2 = lambda i: (0, 0)
    return pl.pallas_call(
        _attn_kernel,
        grid=grid,
        in_specs=[
            pl.BlockSpec((BB, N_AGENTS, OBS_SIZE), lambda i: (i, 0, 0)),
            pl.BlockSpec((OBS_SIZE, 3 * OBS_SIZE), full2),
            pl.BlockSpec((1, 3 * OBS_SIZE), full2),
            pl.BlockSpec((VIS_W, SUB), full2),
            pl.BlockSpec((VIS_W, SUB), full2),
            pl.BlockSpec((SUB, SUB), full2),
            pl.BlockSpec((SUB, SUB), full2),
            pl.BlockSpec((SUB, SUB), full2),
        ],
        out_specs=pl.BlockSpec((BB, N_AGENTS, 2 * OBS_SIZE), lambda i: (i, 0, 0)),
        compiler_params=pltpu.CompilerParams(dimension_semantics=("parallel",)),
        out_shape=jax.ShapeDtypeStruct((batch, N_AGENTS, 2 * OBS_SIZE), jnp.float32),
    )(obs, W, b2, selr, sell, tri, validf, base)
